# baseline (speedup 1.0000x reference)
import jax
import jax.numpy as jnp
from jax.experimental import pallas as pl
from jax.experimental.pallas import tpu as pltpu

NUM_VECTOR = 8
LAYER_IDX = 16


def _body(x_ref, w_ref, b_ref, pool_ref, o_ref,
          tail_ref, big_sem, in_sem, out_sem):
    B, S, H = x_ref.shape
    T = 8
    big = [pltpu.make_async_copy(x_ref.at[b, pl.ds(0, S - T)],
                                 o_ref.at[b, pl.ds(0, S - T)],
                                 big_sem.at[b]) for b in range(B)]
    for c in big:
        c.start()
    rows_in = [pltpu.make_async_copy(x_ref.at[b, pl.ds(S - T, T)],
                                     tail_ref.at[b],
                                     in_sem.at[b]) for b in range(B)]
    for c in rows_in:
        c.start()
    for c in rows_in:
        c.wait()

    act = tail_ref[:, T - 1, :]
    scores = jax.lax.dot_general(
        act, w_ref[...], (((1,), (0,)), ((), ())),
        precision=jax.lax.Precision.HIGHEST)
    scores = scores + b_ref[...]
    idx = jnp.argmax(scores, axis=1).reshape(-1, 1)
    keep = idx == NUM_VECTOR
    onehot = (jax.lax.broadcasted_iota(jnp.int32, (act.shape[0], NUM_VECTOR), 1)
              == idx).astype(jnp.float32)
    repl = jax.lax.dot_general(
        onehot, pool_ref[...], (((1,), (0,)), ((), ())),
        precision=jax.lax.Precision.HIGHEST)
    tail_ref[:, T - 1, :] = jnp.where(keep, act, repl)

    rows_out = [pltpu.make_async_copy(tail_ref.at[b],
                                      o_ref.at[b, pl.ds(S - T, T)],
                                      out_sem.at[b]) for b in range(B)]
    for c in rows_out:
        c.start()
    for c in rows_out:
        c.wait()
    for c in big:
        c.wait()


def kernel(x, vector_pool, gate_W, gate_b):
    B, S, H = x.shape
    pool_layer = vector_pool[:, LAYER_IDX, :]
    gate_b2 = gate_b.reshape(1, -1)
    return pl.pallas_call(
        _body,
        in_specs=[
            pl.BlockSpec(memory_space=pltpu.MemorySpace.HBM),
            pl.BlockSpec(memory_space=pltpu.MemorySpace.VMEM),
            pl.BlockSpec(memory_space=pltpu.MemorySpace.VMEM),
            pl.BlockSpec(memory_space=pltpu.MemorySpace.VMEM),
        ],
        out_specs=pl.BlockSpec(memory_space=pltpu.MemorySpace.HBM),
        out_shape=jax.ShapeDtypeStruct((B, S, H), x.dtype),
        scratch_shapes=[
            pltpu.VMEM((B, 8, H), jnp.float32),
            pltpu.SemaphoreType.DMA((B,)),
            pltpu.SemaphoreType.DMA((B,)),
            pltpu.SemaphoreType.DMA((B,)),
        ],
    )(x, gate_W, gate_b2, pool_layer)


# speedup vs baseline: 46.4221x; 46.4221x over previous
import functools

import jax
import jax.numpy as jnp
from jax.experimental import pallas as pl
from jax.experimental.pallas import tpu as pltpu

NUM_VECTOR = 8
LAYER_IDX = 16
BLK = 1024


def _body(nblk, x_ref, w_ref, b_ref, pool_ref, o_ref):
    s = pl.program_id(1)
    o_ref[0] = x_ref[0]

    @pl.when(s == nblk - 1)
    def _():
        act = x_ref[0, pl.ds(BLK - 1, 1), :]
        scores = jax.lax.dot_general(
            act, w_ref[...], (((1,), (0,)), ((), ())),
            precision=jax.lax.Precision.HIGHEST)
        scores = scores + b_ref[...]
        idx = jnp.argmax(scores[0, :], axis=0)
        keep = idx == NUM_VECTOR
        onehot = (jax.lax.broadcasted_iota(jnp.int32, (1, NUM_VECTOR), 1)
                  == idx).astype(jnp.float32)
        repl = jax.lax.dot_general(
            onehot, pool_ref[...], (((1,), (0,)), ((), ())),
            precision=jax.lax.Precision.HIGHEST)
        o_ref[0, pl.ds(BLK - 1, 1), :] = jnp.where(keep, act, repl)


def kernel(x, vector_pool, gate_W, gate_b):
    B, S, H = x.shape
    nblk = S // BLK
    pool_layer = vector_pool[:, LAYER_IDX, :]
    gate_b2 = gate_b.reshape(1, -1)
    grid = (B, nblk)
    return pl.pallas_call(
        functools.partial(_body, nblk),
        grid=grid,
        in_specs=[
            pl.BlockSpec((1, BLK, H), lambda b, s: (b, s, 0)),
            pl.BlockSpec((H, NUM_VECTOR + 1), lambda b, s: (0, 0)),
            pl.BlockSpec((1, NUM_VECTOR + 1), lambda b, s: (0, 0)),
            pl.BlockSpec((NUM_VECTOR, H), lambda b, s: (0, 0)),
        ],
        out_specs=pl.BlockSpec((1, BLK, H), lambda b, s: (b, s, 0)),
        out_shape=jax.ShapeDtypeStruct((B, S, H), x.dtype),
        compiler_params=pltpu.CompilerParams(
            dimension_semantics=("parallel", "arbitrary")),
    )(x, gate_W, gate_b2, pool_layer)
